# baseline (device time: 160232 ns/iter reference)
import jax
import jax.numpy as jnp
from jax import lax
from jax.experimental import pallas as pl
from jax.experimental.pallas import tpu as pltpu

N_DEV = 4


def kernel(x, w_mat, scale_x, scale_w):
    if x.dtype != jnp.float8_e4m3fn:
        x = x.astype(jnp.float8_e4m3fn)
    if w_mat.dtype != jnp.float8_e4m3fn:
        w_mat = w_mat.astype(jnp.float8_e4m3fn)
    m_per, k = x.shape
    n_per = w_mat.shape[1]
    s = (scale_x.astype(jnp.float32) * scale_w.astype(jnp.float32)).reshape(1, 1)

    def body(x_ref, w_ref, s_ref, out_ref, comm_ref, send_sems, recv_sems):
        my = lax.axis_index("i")
        left = lax.rem(my + (N_DEV - 1), N_DEV)
        right = lax.rem(my + 1, N_DEV)

        barrier_sem = pltpu.get_barrier_semaphore()
        for nbr in (left, right):
            pl.semaphore_signal(
                barrier_sem, inc=1,
                device_id=(nbr,), device_id_type=pl.DeviceIdType.MESH,
            )
        pl.semaphore_wait(barrier_sem, 2)

        comm_ref[0] = x_ref[...]
        scale = s_ref[0, 0]

        def compute(chunk, origin):
            acc = jnp.dot(chunk, w_ref[...], preferred_element_type=jnp.float32)
            out_ref[pl.ds(origin * m_per, m_per), :] = jnp.maximum(acc * scale, 0.0)

        for h in range(N_DEV - 1):
            send_slot = h % 2
            recv_slot = (h + 1) % 2
            rdma = pltpu.make_async_remote_copy(
                src_ref=comm_ref.at[send_slot],
                dst_ref=comm_ref.at[recv_slot],
                send_sem=send_sems.at[send_slot],
                recv_sem=recv_sems.at[recv_slot],
                device_id=(right,),
                device_id_type=pl.DeviceIdType.MESH,
            )
            rdma.start()
            origin = lax.rem(my - h + N_DEV, N_DEV)
            compute(comm_ref[send_slot], origin)
            rdma.wait()
        compute(comm_ref[(N_DEV - 1) % 2], lax.rem(my + 1, N_DEV))

    out_shape = jax.ShapeDtypeStruct((N_DEV * m_per, n_per), jnp.float32)
    return pl.pallas_call(
        body,
        out_shape=out_shape,
        in_specs=[
            pl.BlockSpec(memory_space=pltpu.VMEM),
            pl.BlockSpec(memory_space=pltpu.VMEM),
            pl.BlockSpec(memory_space=pltpu.SMEM),
        ],
        out_specs=pl.BlockSpec(memory_space=pltpu.VMEM),
        scratch_shapes=[
            pltpu.VMEM((2, m_per, k), jnp.float8_e4m3fn),
            pltpu.SemaphoreType.DMA((2,)),
            pltpu.SemaphoreType.DMA((2,)),
        ],
        compiler_params=pltpu.CompilerParams(collective_id=0),
    )(x, w_mat, s)


# device time: 93000 ns/iter; 1.7229x vs baseline; 1.7229x over previous
import jax
import jax.numpy as jnp
from jax import lax
from jax.experimental import pallas as pl
from jax.experimental.pallas import tpu as pltpu

N_DEV = 4


def kernel(x, w_mat, scale_x, scale_w):
    if x.dtype != jnp.float8_e4m3fn:
        x = x.astype(jnp.float8_e4m3fn)
    if w_mat.dtype != jnp.float8_e4m3fn:
        w_mat = w_mat.astype(jnp.float8_e4m3fn)
    m_per, k = x.shape
    m_half = m_per // 2
    n_per = w_mat.shape[1]
    s = (scale_x.astype(jnp.float32) * scale_w.astype(jnp.float32)).reshape(1, 1)

    def body(x_ref, w_ref, s_ref, out_ref, comm_cw, comm_ccw,
             send_cw, recv_cw, send_ccw, recv_ccw):
        my = lax.axis_index("i")
        left = lax.rem(my + (N_DEV - 1), N_DEV)
        right = lax.rem(my + 1, N_DEV)

        barrier_sem = pltpu.get_barrier_semaphore()
        for nbr in (left, right):
            pl.semaphore_signal(
                barrier_sem, inc=1,
                device_id=(nbr,), device_id_type=pl.DeviceIdType.MESH,
            )
        pl.semaphore_wait(barrier_sem, 2)

        comm_cw[0] = x_ref[:m_half, :]
        comm_ccw[0] = x_ref[m_half:, :]
        scale = s_ref[0, 0]

        def store(rows, origin, half):
            acc = jnp.dot(rows, w_ref[...], preferred_element_type=jnp.float32)
            off = origin * m_per + half * m_half
            out_ref[pl.ds(off, m_half), :] = jnp.maximum(acc * scale, 0.0)

        for h in range(N_DEV - 1):
            ss = h % 2
            rs = (h + 1) % 2
            rdma_cw = pltpu.make_async_remote_copy(
                src_ref=comm_cw.at[ss], dst_ref=comm_cw.at[rs],
                send_sem=send_cw.at[ss], recv_sem=recv_cw.at[rs],
                device_id=(right,), device_id_type=pl.DeviceIdType.MESH,
            )
            rdma_ccw = pltpu.make_async_remote_copy(
                src_ref=comm_ccw.at[ss], dst_ref=comm_ccw.at[rs],
                send_sem=send_ccw.at[ss], recv_sem=recv_ccw.at[rs],
                device_id=(left,), device_id_type=pl.DeviceIdType.MESH,
            )
            rdma_cw.start()
            rdma_ccw.start()
            if h == 0:
                store(x_ref[:m_half, :], my, 0)
                store(x_ref[m_half:, :], my, 1)
            else:
                store(comm_cw[ss], lax.rem(my - h + N_DEV, N_DEV), 0)
                store(comm_ccw[ss], lax.rem(my + h, N_DEV), 1)
            rdma_cw.wait()
            rdma_ccw.wait()

        last = (N_DEV - 1) % 2
        store(comm_cw[last], lax.rem(my + 1, N_DEV), 0)
        store(comm_ccw[last], lax.rem(my + N_DEV - 1, N_DEV), 1)

    out_shape = jax.ShapeDtypeStruct((N_DEV * m_per, n_per), jnp.float32)
    return pl.pallas_call(
        body,
        out_shape=out_shape,
        in_specs=[
            pl.BlockSpec(memory_space=pltpu.VMEM),
            pl.BlockSpec(memory_space=pltpu.VMEM),
            pl.BlockSpec(memory_space=pltpu.SMEM),
        ],
        out_specs=pl.BlockSpec(memory_space=pltpu.VMEM),
        scratch_shapes=[
            pltpu.VMEM((2, m_half, k), jnp.float8_e4m3fn),
            pltpu.VMEM((2, m_half, k), jnp.float8_e4m3fn),
            pltpu.SemaphoreType.DMA((2,)),
            pltpu.SemaphoreType.DMA((2,)),
            pltpu.SemaphoreType.DMA((2,)),
            pltpu.SemaphoreType.DMA((2,)),
        ],
        compiler_params=pltpu.CompilerParams(collective_id=0),
    )(x, w_mat, s)


# device time: 91993 ns/iter; 1.7418x vs baseline; 1.0109x over previous
import jax
import jax.numpy as jnp
from jax import lax
from jax.experimental import pallas as pl
from jax.experimental.pallas import tpu as pltpu

N_DEV = 4
FP8 = jnp.float8_e4m3fn


def kernel(x, w_mat, scale_x, scale_w):
    m_per, k = x.shape
    m_half = m_per // 2
    n_per = w_mat.shape[1]
    s = (scale_x.astype(jnp.float32) * scale_w.astype(jnp.float32)).reshape(1, 1)

    def body(x_ref, w_ref, s_ref, out_ref, w8, comm_cw, comm_ccw,
             send_cw, recv_cw, send_ccw, recv_ccw):
        my = lax.axis_index("i")
        left = lax.rem(my + (N_DEV - 1), N_DEV)
        right = lax.rem(my + 1, N_DEV)

        barrier_sem = pltpu.get_barrier_semaphore()
        for nbr in (left, right):
            pl.semaphore_signal(
                barrier_sem, inc=1,
                device_id=(nbr,), device_id_type=pl.DeviceIdType.MESH,
            )
        pl.semaphore_wait(barrier_sem, 2)

        comm_cw[0] = x_ref[:m_half, :].astype(FP8)
        comm_ccw[0] = x_ref[m_half:, :].astype(FP8)
        w8[...] = w_ref[...].astype(FP8)
        scale = s_ref[0, 0]

        def store(rows, origin, half):
            acc = jnp.dot(rows, w8[...], preferred_element_type=jnp.float32)
            off = origin * m_per + half * m_half
            out_ref[pl.ds(off, m_half), :] = jnp.maximum(acc * scale, 0.0)

        for h in range(N_DEV - 1):
            ss = h % 2
            rs = (h + 1) % 2
            rdma_cw = pltpu.make_async_remote_copy(
                src_ref=comm_cw.at[ss], dst_ref=comm_cw.at[rs],
                send_sem=send_cw.at[ss], recv_sem=recv_cw.at[rs],
                device_id=(right,), device_id_type=pl.DeviceIdType.MESH,
            )
            rdma_ccw = pltpu.make_async_remote_copy(
                src_ref=comm_ccw.at[ss], dst_ref=comm_ccw.at[rs],
                send_sem=send_ccw.at[ss], recv_sem=recv_ccw.at[rs],
                device_id=(left,), device_id_type=pl.DeviceIdType.MESH,
            )
            rdma_cw.start()
            rdma_ccw.start()
            store(comm_cw[ss], lax.rem(my - h + N_DEV, N_DEV), 0)
            store(comm_ccw[ss], lax.rem(my + h, N_DEV), 1)
            rdma_cw.wait()
            rdma_ccw.wait()

        last = (N_DEV - 1) % 2
        store(comm_cw[last], lax.rem(my + 1, N_DEV), 0)
        store(comm_ccw[last], lax.rem(my + N_DEV - 1, N_DEV), 1)

    out_shape = jax.ShapeDtypeStruct((N_DEV * m_per, n_per), jnp.float32)
    return pl.pallas_call(
        body,
        out_shape=out_shape,
        in_specs=[
            pl.BlockSpec(memory_space=pltpu.VMEM),
            pl.BlockSpec(memory_space=pltpu.VMEM),
            pl.BlockSpec(memory_space=pltpu.SMEM),
        ],
        out_specs=pl.BlockSpec(memory_space=pltpu.VMEM),
        scratch_shapes=[
            pltpu.VMEM((k, n_per), FP8),
            pltpu.VMEM((2, m_half, k), FP8),
            pltpu.VMEM((2, m_half, k), FP8),
            pltpu.SemaphoreType.DMA((2,)),
            pltpu.SemaphoreType.DMA((2,)),
            pltpu.SemaphoreType.DMA((2,)),
            pltpu.SemaphoreType.DMA((2,)),
        ],
        compiler_params=pltpu.CompilerParams(collective_id=0),
    )(x, w_mat, s)


# device time: 86593 ns/iter; 1.8504x vs baseline; 1.0624x over previous
import jax
import jax.numpy as jnp
from jax import lax
from jax.experimental import pallas as pl
from jax.experimental.pallas import tpu as pltpu

N_DEV = 4
NQ = 2
FP8 = jnp.float8_e4m3fn


def kernel(x, w_mat, scale_x, scale_w):
    m_per, k = x.shape
    m_half = m_per // 2
    m_q = m_half // NQ
    n_per = w_mat.shape[1]
    s = (scale_x.astype(jnp.float32) * scale_w.astype(jnp.float32)).reshape(1, 1)

    def body(x_ref, w_ref, s_ref, out_ref, w8, comm_cw, comm_ccw,
             send_cw, recv_cw, send_ccw, recv_ccw):
        my = lax.axis_index("i")
        left = lax.rem(my + (N_DEV - 1), N_DEV)
        right = lax.rem(my + 1, N_DEV)

        barrier_sem = pltpu.get_barrier_semaphore()
        for nbr in (left, right):
            pl.semaphore_signal(
                barrier_sem, inc=1,
                device_id=(nbr,), device_id_type=pl.DeviceIdType.MESH,
            )
        pl.semaphore_wait(barrier_sem, 2)

        def rdma(comm, ssem, rsem, q, h, target):
            return pltpu.make_async_remote_copy(
                src_ref=comm.at[q, h % 2],
                dst_ref=comm.at[q, (h + 1) % 2],
                send_sem=ssem.at[q, h % 2],
                recv_sem=rsem.at[q, (h + 1) % 2],
                device_id=(target,), device_id_type=pl.DeviceIdType.MESH,
            )

        rdma_cw = lambda q, h: rdma(comm_cw, send_cw, recv_cw, q, h, right)
        rdma_ccw = lambda q, h: rdma(comm_ccw, send_ccw, recv_ccw, q, h, left)

        for q in range(NQ):
            comm_cw[q, 0] = x_ref[pl.ds(q * m_q, m_q), :].astype(FP8)
            rdma_cw(q, 0).start()
            comm_ccw[q, 0] = x_ref[pl.ds(m_half + q * m_q, m_q), :].astype(FP8)
            rdma_ccw(q, 0).start()
        w8[...] = w_ref[...].astype(FP8)
        scale = s_ref[0, 0]

        def store(rows, origin, half, q):
            acc = jnp.dot(rows, w8[...], preferred_element_type=jnp.float32)
            off = origin * m_per + half * m_half + q * m_q
            out_ref[pl.ds(off, m_q), :] = jnp.maximum(acc * scale, 0.0)

        for q in range(NQ):
            store(comm_cw[q, 0], my, 0, q)
            store(comm_ccw[q, 0], my, 1, q)

        for h in range(N_DEV - 1):
            rs = (h + 1) % 2
            for q in range(NQ):
                for fwd, recv_of, origin, half in (
                    (rdma_cw, rdma_cw, lax.rem(my - h - 1 + N_DEV, N_DEV), 0),
                    (rdma_ccw, rdma_ccw, lax.rem(my + h + 1, N_DEV), 1),
                ):
                    recv_of(q, h).wait_recv()
                    if h < N_DEV - 2:
                        if h >= 1:
                            fwd(q, h - 1).wait_send()
                        fwd(q, h + 1).start()
                    store(comm_cw[q, rs] if half == 0 else comm_ccw[q, rs],
                          origin, half, q)

        for q in range(NQ):
            for h in (N_DEV - 3, N_DEV - 2):
                rdma_cw(q, h).wait_send()
                rdma_ccw(q, h).wait_send()

    out_shape = jax.ShapeDtypeStruct((N_DEV * m_per, n_per), jnp.float32)
    return pl.pallas_call(
        body,
        out_shape=out_shape,
        in_specs=[
            pl.BlockSpec(memory_space=pltpu.VMEM),
            pl.BlockSpec(memory_space=pltpu.VMEM),
            pl.BlockSpec(memory_space=pltpu.SMEM),
        ],
        out_specs=pl.BlockSpec(memory_space=pltpu.VMEM),
        scratch_shapes=[
            pltpu.VMEM((k, n_per), FP8),
            pltpu.VMEM((NQ, 2, m_q, k), FP8),
            pltpu.VMEM((NQ, 2, m_q, k), FP8),
            pltpu.SemaphoreType.DMA((NQ, 2)),
            pltpu.SemaphoreType.DMA((NQ, 2)),
            pltpu.SemaphoreType.DMA((NQ, 2)),
            pltpu.SemaphoreType.DMA((NQ, 2)),
        ],
        compiler_params=pltpu.CompilerParams(collective_id=0),
    )(x, w_mat, s)
